# baseline (speedup 1.0000x reference)
import functools

import jax
import jax.numpy as jnp
from jax import lax
from jax.experimental import pallas as pl
from jax.experimental.pallas import tpu as pltpu
from jax.experimental.pallas import tpu_sc as plsc

N_NODES = 10000
N_EDGES = 320000
DIM = 128
EDGE_NUM = 3000

_NC = 2
_NS = 16
_NW = _NC * _NS
_EW = N_EDGES // _NW
_C = 80
_NCH = _EW // _C
_CV = _C // 16


def _etype(s_t, d_t):
    diff = jnp.abs(s_t - d_t) - 1
    return s_t * d_t + (diff * diff) // 4


_NB = 5


def _edge_emb_kernel(node_hbm, edge_hbm, emb_hbm, out_hbm,
                     node_v, src_v, dst_v, et_v, rows_v,
                     sem_in, gsem, ssem):
    wid = lax.axis_index("s") * _NC + lax.axis_index("c")
    base = wid * _EW

    cp_n = pltpu.async_copy(node_hbm, node_v, sem_in)
    cp_s = pltpu.async_copy(edge_hbm.at[pl.ds(base, _EW)], src_v, sem_in)
    cp_d = pltpu.async_copy(edge_hbm.at[pl.ds(N_EDGES + base, _EW)], dst_v,
                            sem_in)
    cp_n.wait()
    cp_s.wait()
    cp_d.wait()

    def compute_row(j):
        for k in range(_CV):
            off = j * _C + k * 16
            si = src_v[pl.ds(off, 16)]
            di = dst_v[pl.ds(off, 16)]
            s_t = plsc.load_gather(node_v, [si])
            d_t = plsc.load_gather(node_v, [di])
            et_v[j, pl.ds(k * 16, 16)] = _etype(s_t, d_t)

    def fire_g(j, b):
        pltpu.async_copy(emb_hbm.at[et_v.at[j]], rows_v.at[b], gsem.at[b])

    def wait_g(j, b):
        pltpu.make_async_copy(emb_hbm.at[et_v.at[j]], rows_v.at[b],
                              gsem.at[b]).wait()

    def fire_s(j, b):
        pltpu.async_copy(rows_v.at[b], out_hbm.at[pl.ds(base + j * _C, _C)],
                         ssem.at[b])

    def wait_s(j, b):
        pltpu.make_async_copy(rows_v.at[b],
                              out_hbm.at[pl.ds(base + j * _C, _C)],
                              ssem.at[b]).wait()

    for b in range(_NB):
        compute_row(b)
        fire_g(b, b)

    def group(g, _):
        j0 = g * _NB
        for b in range(_NB):
            compute_row(j0 + _NB + b)
        for b in range(_NB):
            wait_g(j0 + b, b)
            fire_s(j0 + b, b)
        for b in range(_NB):
            wait_s(j0 + b, b)
            fire_g(j0 + _NB + b, b)
        return 0

    lax.fori_loop(0, _NCH // _NB - 1, group, 0)
    j0 = _NCH - _NB
    for b in range(_NB):
        wait_g(j0 + b, b)
        fire_s(j0 + b, b)
    for b in range(_NB):
        wait_s(j0 + b, b)


@jax.jit
def _edge_emb(node_type, edge_index, emb):
    mesh = plsc.VectorSubcoreMesh(core_axis_name="c", subcore_axis_name="s",
                                  num_cores=_NC, num_subcores=_NS)
    return pl.kernel(
        _edge_emb_kernel,
        out_type=jax.ShapeDtypeStruct((N_EDGES, DIM), jnp.float32),
        mesh=mesh,
        scratch_types=[
            pltpu.VMEM((N_NODES,), jnp.int32),
            pltpu.VMEM((_EW,), jnp.int32),
            pltpu.VMEM((_EW,), jnp.int32),
            pltpu.VMEM((_NCH, _C), jnp.int32),
            pltpu.VMEM((_NB, _C, DIM), jnp.float32),
            pltpu.SemaphoreType.DMA,
            pltpu.SemaphoreType.DMA((_NB,)),
            pltpu.SemaphoreType.DMA((_NB,)),
        ],
        compiler_params=pltpu.CompilerParams(needs_layout_passes=False),
    )(node_type, edge_index.reshape(2 * N_EDGES), emb)


def kernel(node_type, edge_index, emb):
    return _edge_emb(node_type, edge_index, emb)


# speedup vs baseline: 1.0184x; 1.0184x over previous
import functools

import jax
import jax.numpy as jnp
from jax import lax
from jax.experimental import pallas as pl
from jax.experimental.pallas import tpu as pltpu
from jax.experimental.pallas import tpu_sc as plsc

N_NODES = 10000
N_EDGES = 320000
DIM = 128
EDGE_NUM = 3000

_NC = 2
_NS = 16
_NW = _NC * _NS
_EW = N_EDGES // _NW
_C = 80
_NCH = _EW // _C
_CV = _C // 16


def _etype(s_t, d_t):
    diff = jnp.abs(s_t - d_t) - 1
    return s_t * d_t + (diff * diff) // 4


_NB = 5


def _edge_emb_kernel(node_hbm, edge_hbm, emb_hbm, out_hbm,
                     node_v, src_v, dst_v, et_v, rows_v,
                     sem_in, gsem, ssem):
    wid = lax.axis_index("s") * _NC + lax.axis_index("c")
    base = wid * _EW

    cp_n = pltpu.async_copy(node_hbm, node_v, sem_in)
    cp_s = pltpu.async_copy(edge_hbm.at[pl.ds(base, _EW)], src_v, sem_in)
    cp_d = pltpu.async_copy(edge_hbm.at[pl.ds(N_EDGES + base, _EW)], dst_v,
                            sem_in)
    cp_n.wait()
    cp_s.wait()
    cp_d.wait()

    def compute_row(j):
        for k in range(_CV):
            off = j * _C + k * 16
            si = src_v[pl.ds(off, 16)]
            di = dst_v[pl.ds(off, 16)]
            s_t = plsc.load_gather(node_v, [si])
            d_t = plsc.load_gather(node_v, [di])
            et_v[j, pl.ds(k * 16, 16)] = _etype(s_t, d_t)

    def fire_g(j, b):
        pltpu.async_copy(emb_hbm.at[et_v.at[j]], rows_v.at[b], gsem.at[b])

    def wait_g(j, b):
        pltpu.make_async_copy(emb_hbm.at[et_v.at[j]], rows_v.at[b],
                              gsem.at[b]).wait()

    def fire_s(j, b):
        pltpu.async_copy(rows_v.at[b], out_hbm.at[pl.ds(base + j * _C, _C)],
                         ssem.at[b])

    def wait_s(j, b):
        pltpu.make_async_copy(rows_v.at[b],
                              out_hbm.at[pl.ds(base + j * _C, _C)],
                              ssem.at[b]).wait()

    for b in range(_NB):
        compute_row(b)
        fire_g(b, b)

    def group(g, _):
        j0 = g * _NB
        for b in range(_NB):
            compute_row(j0 + _NB + b)
        for b in range(_NB):
            wait_g(j0 + b, b)
            fire_s(j0 + b, b)
            wait_s(j0 + b, b)
            fire_g(j0 + _NB + b, b)
        return 0

    lax.fori_loop(0, _NCH // _NB - 1, group, 0)
    j0 = _NCH - _NB
    for b in range(_NB):
        wait_g(j0 + b, b)
        fire_s(j0 + b, b)
        wait_s(j0 + b, b)


@jax.jit
def _edge_emb(node_type, edge_index, emb):
    mesh = plsc.VectorSubcoreMesh(core_axis_name="c", subcore_axis_name="s",
                                  num_cores=_NC, num_subcores=_NS)
    return pl.kernel(
        _edge_emb_kernel,
        out_type=jax.ShapeDtypeStruct((N_EDGES, DIM), jnp.float32),
        mesh=mesh,
        scratch_types=[
            pltpu.VMEM((N_NODES,), jnp.int32),
            pltpu.VMEM((_EW,), jnp.int32),
            pltpu.VMEM((_EW,), jnp.int32),
            pltpu.VMEM((_NCH, _C), jnp.int32),
            pltpu.VMEM((_NB, _C, DIM), jnp.float32),
            pltpu.SemaphoreType.DMA,
            pltpu.SemaphoreType.DMA((_NB,)),
            pltpu.SemaphoreType.DMA((_NB,)),
        ],
        compiler_params=pltpu.CompilerParams(needs_layout_passes=False),
    )(node_type, edge_index.reshape(2 * N_EDGES), emb)


def kernel(node_type, edge_index, emb):
    return _edge_emb(node_type, edge_index, emb)


# speedup vs baseline: 1.0196x; 1.0012x over previous
import functools

import jax
import jax.numpy as jnp
from jax import lax
from jax.experimental import pallas as pl
from jax.experimental.pallas import tpu as pltpu
from jax.experimental.pallas import tpu_sc as plsc

N_NODES = 10000
N_EDGES = 320000
DIM = 128
EDGE_NUM = 3000

_NC = 2
_NS = 16
_NW = _NC * _NS
_EW = N_EDGES // _NW
_C = 80
_NCH = _EW // _C
_CV = _C // 16


def _etype(s_t, d_t):
    diff = jnp.abs(s_t - d_t) - 1
    return s_t * d_t + (diff * diff) // 4


_NB = 5


def _edge_emb_kernel(node_hbm, edge_hbm, emb_hbm, out_hbm,
                     node_v, src_v, dst_v, et_v, rows_v,
                     sem_in, gsem, ssem):
    wid = lax.axis_index("s") * _NC + lax.axis_index("c")
    base = wid * _EW

    cp_n = pltpu.async_copy(node_hbm, node_v, sem_in)
    cp_s = pltpu.async_copy(edge_hbm.at[pl.ds(base, _EW)], src_v, sem_in)
    cp_d = pltpu.async_copy(edge_hbm.at[pl.ds(N_EDGES + base, _EW)], dst_v,
                            sem_in)
    cp_n.wait()
    cp_s.wait()
    cp_d.wait()

    def compute_row(j):
        for k in range(_CV):
            off = j * _C + k * 16
            si = src_v[pl.ds(off, 16)]
            di = dst_v[pl.ds(off, 16)]
            s_t = plsc.load_gather(node_v, [si])
            d_t = plsc.load_gather(node_v, [di])
            et_v[j, pl.ds(k * 16, 16)] = _etype(s_t, d_t)

    def fire_g(j, b):
        pltpu.async_copy(emb_hbm.at[et_v.at[j]], rows_v.at[b], gsem.at[b])

    def wait_g(j, b):
        pltpu.make_async_copy(emb_hbm.at[et_v.at[j]], rows_v.at[b],
                              gsem.at[b]).wait()

    def fire_s(j, b):
        pltpu.async_copy(rows_v.at[b], out_hbm.at[pl.ds(base + j * _C, _C)],
                         ssem.at[b])

    def wait_s(j, b):
        pltpu.make_async_copy(rows_v.at[b],
                              out_hbm.at[pl.ds(base + j * _C, _C)],
                              ssem.at[b]).wait()

    for b in range(_NB):
        compute_row(b)
        fire_g(b, b)

    def group(g, _):
        j0 = g * _NB
        for b in range(_NB):
            compute_row(j0 + _NB + b)
        for b in range(_NB):
            wait_g(j0 + b, b)
            fire_s(j0 + b, b)
            wait_s(j0 + b, b)
            fire_g(j0 + _NB + b, b)
        return 0

    lax.fori_loop(0, _NCH // _NB - 1, group, 0)
    j0 = _NCH - _NB
    for b in range(_NB):
        wait_g(j0 + b, b)
        fire_s(j0 + b, b)
        wait_s(j0 + b, b)


@jax.jit
def _edge_emb(node_type, edge_index, emb):
    mesh = plsc.VectorSubcoreMesh(core_axis_name="c", subcore_axis_name="s",
                                  num_cores=_NC, num_subcores=_NS)
    return pl.kernel(
        _edge_emb_kernel,
        out_type=jax.ShapeDtypeStruct((N_EDGES, DIM), jnp.float32),
        mesh=mesh,
        scratch_types=[
            pltpu.VMEM((N_NODES,), jnp.int32),
            pltpu.VMEM((_EW,), jnp.int32),
            pltpu.VMEM((_EW,), jnp.int32),
            pltpu.VMEM((_NCH, _C), jnp.int32),
            pltpu.VMEM((_NB, _C, DIM), jnp.float32),
            pltpu.SemaphoreType.DMA,
            pltpu.SemaphoreType.DMA((_NB,)),
            pltpu.SemaphoreType.DMA((_NB,)),
        ],
        compiler_params=pltpu.CompilerParams(needs_layout_passes=False, skip_device_barrier=True),
    )(node_type, edge_index.reshape(2 * N_EDGES), emb)


def kernel(node_type, edge_index, emb):
    return _edge_emb(node_type, edge_index, emb)
